# baseline (speedup 1.0000x reference)
import functools
import jax
import jax.numpy as jnp
from jax import lax
from jax.experimental import pallas as pl
from jax.experimental.pallas import tpu as pltpu, tpu_sc as plsc


def _make_sc_kernel(b, d, h, w):
    mesh = plsc.VectorSubcoreMesh(core_axis_name="c", subcore_axis_name="s")
    n_ch = 16
    hw = h * w

    @functools.partial(
        pl.kernel,
        out_type=jax.ShapeDtypeStruct((b, 2 * d, hw), jnp.float32),
        mesh=mesh,
        scratch_types=[
            pltpu.VMEM((2 * h, d), jnp.float32),
            pltpu.VMEM((n_ch, hw), jnp.float32),
            pltpu.SemaphoreType.DMA,
        ],
        compiler_params=pltpu.CompilerParams(needs_layout_passes=False),
    )
    def k(row_hbm, col_hbm, out_hbm, t, buf, sem):
        nc = 2
        wid = lax.axis_index("s") * nc + lax.axis_index("c")
        wid16 = lax.rem(wid, 16)
        is_row = wid >= 16
        c0 = n_ch * wid16

        pltpu.sync_copy(col_hbm.at[pl.ds(0, h)], t.at[pl.ds(0, h)])
        pltpu.sync_copy(row_hbm.at[pl.ds(0, h)], t.at[pl.ds(h, h)])

        iota = lax.broadcasted_iota(jnp.int32, (16,), 0)

        def cc_body(cc, carry):
            ccv = jnp.full((16,), c0 + cc, jnp.int32)
            for i in range(h):
                rows_row = jnp.full((16,), h + i, jnp.int32)
                for jb in range(w // 16):
                    rows_col = iota + 16 * jb
                    rows = jnp.where(is_row, rows_row, rows_col)
                    v = plsc.load_gather(t, [rows, ccv])
                    buf[cc, pl.ds(i * w + 16 * jb, 16)] = v
            return carry

        lax.fori_loop(0, n_ch, cc_body, 0)

        out_c = jnp.where(is_row, d + c0, c0)
        descs = [
            pltpu.async_copy(buf, out_hbm.at[bi, pl.ds(out_c, n_ch)], sem)
            for bi in range(b)
        ]
        for de in descs:
            de.wait()

    return k


def kernel(x, row_embed, col_embed):
    b = x.shape[0]
    h, w = x.shape[-2], x.shape[-1]
    d = col_embed.shape[-1]
    out = _make_sc_kernel(b, d, h, w)(row_embed, col_embed)
    return out.reshape(b, 2 * d, h, w)
